# baseline (speedup 1.0000x reference)
import jax
import jax.numpy as jnp
from jax import lax
from jax.experimental import pallas as pl
from jax.experimental.pallas import tpu as pltpu
from jax.experimental.pallas import tpu_sc as plsc

B = 8
V = 16
NIN = 4
NTGT = 8
IMG_W = 3 * 256 * 256
POSE_W = 16

_mesh = plsc.VectorSubcoreMesh(core_axis_name="c", subcore_axis_name="s")


def _sc_split(image2d, pose2d, idxflat):
    def body(img_ref, pose_ref, idx_hbm, in_img, tgt_img, in_pose, tgt_pose,
             idx_v):
        cid = lax.axis_index("c")
        sid = lax.axis_index("s")
        w = sid * 2 + cid

        pltpu.sync_copy(idx_hbm, idx_v.at[pl.ds(0, B * NTGT)])

        b_in = w // 4
        v_in = w % 4
        pltpu.sync_copy(img_ref.at[b_in * V + v_in], in_img.at[w])
        pltpu.sync_copy(pose_ref.at[b_in * V + v_in], in_pose.at[w])

        e0 = 2 * w
        vec = idx_v[pl.ds(e0, 16)]
        v0 = vec[0]
        v1 = vec[1]
        b0 = e0 // NTGT
        b1 = (e0 + 1) // NTGT
        pltpu.sync_copy(img_ref.at[b0 * V + v0], tgt_img.at[e0])
        pltpu.sync_copy(img_ref.at[b1 * V + v1], tgt_img.at[e0 + 1])
        pltpu.sync_copy(pose_ref.at[b0 * V + v0], tgt_pose.at[e0])
        pltpu.sync_copy(pose_ref.at[b1 * V + v1], tgt_pose.at[e0 + 1])

    f = pl.kernel(
        body,
        out_type=[
            jax.ShapeDtypeStruct((B * NIN, IMG_W), jnp.float32),
            jax.ShapeDtypeStruct((B * NTGT, IMG_W), jnp.float32),
            jax.ShapeDtypeStruct((B * NIN, POSE_W), jnp.float32),
            jax.ShapeDtypeStruct((B * NTGT, POSE_W), jnp.float32),
        ],
        mesh=_mesh,
        scratch_types=[
            pltpu.VMEM((B * NTGT + 16,), jnp.int32),
        ],
    )
    return f(image2d, pose2d, idxflat)


def kernel(image, pose, index):
    image2d = image.reshape(B * V, IMG_W)
    pose2d = pose.reshape(B * V, POSE_W)
    idxflat = index.reshape(B * NTGT).astype(jnp.int32)
    in_img, tgt_img, in_pose, tgt_pose = _sc_split(image2d, pose2d, idxflat)
    return (
        in_img.reshape(B, NIN, 3, 256, 256),
        in_pose.reshape(B, NIN, 4, 4),
        tgt_img.reshape(B, NTGT, 3, 256, 256),
        tgt_pose.reshape(B, NTGT, 4, 4),
    )


# speedup vs baseline: 1.0022x; 1.0022x over previous
import jax
import jax.numpy as jnp
from jax import lax
from jax.experimental import pallas as pl
from jax.experimental.pallas import tpu as pltpu
from jax.experimental.pallas import tpu_sc as plsc

B = 8
V = 16
NIN = 4
NTGT = 8
IMG_W = 3 * 256 * 256
POSE_W = 16
NCH = 4
CH = IMG_W // NCH

_mesh = plsc.VectorSubcoreMesh(core_axis_name="c", subcore_axis_name="s")


def _sc_split(image2d, pose2d, idxflat):
    def body(img_ref, pose_ref, idx_hbm, in_img, tgt_img, in_pose, tgt_pose,
             idx_v, sem):
        cid = lax.axis_index("c")
        sid = lax.axis_index("s")
        w = sid * 2 + cid

        pltpu.sync_copy(idx_hbm, idx_v.at[pl.ds(0, B * NTGT)])

        e0 = 2 * w
        vec = idx_v[pl.ds(e0, 16)]
        v0 = vec[0]
        v1 = vec[1]
        b0 = e0 // NTGT
        b1 = (e0 + 1) // NTGT
        b_in = w // 4
        v_in = w % 4

        handles = []

        def copy_view(src_row, dst_ref, dst_row):
            for c in range(NCH):
                handles.append(pltpu.async_copy(
                    img_ref.at[src_row, pl.ds(c * CH, CH)],
                    dst_ref.at[dst_row, pl.ds(c * CH, CH)], sem))

        copy_view(b_in * V + v_in, in_img, w)
        copy_view(b0 * V + v0, tgt_img, e0)
        copy_view(b1 * V + v1, tgt_img, e0 + 1)
        handles.append(pltpu.async_copy(
            pose_ref.at[b_in * V + v_in], in_pose.at[w], sem))
        handles.append(pltpu.async_copy(
            pose_ref.at[b0 * V + v0], tgt_pose.at[e0], sem))
        handles.append(pltpu.async_copy(
            pose_ref.at[b1 * V + v1], tgt_pose.at[e0 + 1], sem))
        for h in handles:
            h.wait()

    f = pl.kernel(
        body,
        out_type=[
            jax.ShapeDtypeStruct((B * NIN, IMG_W), jnp.float32),
            jax.ShapeDtypeStruct((B * NTGT, IMG_W), jnp.float32),
            jax.ShapeDtypeStruct((B * NIN, POSE_W), jnp.float32),
            jax.ShapeDtypeStruct((B * NTGT, POSE_W), jnp.float32),
        ],
        mesh=_mesh,
        scratch_types=[
            pltpu.VMEM((B * NTGT + 16,), jnp.int32),
            pltpu.SemaphoreType.DMA,
        ],
    )
    return f(image2d, pose2d, idxflat)


def kernel(image, pose, index):
    image2d = image.reshape(B * V, IMG_W)
    pose2d = pose.reshape(B * V, POSE_W)
    idxflat = index.reshape(B * NTGT).astype(jnp.int32)
    in_img, tgt_img, in_pose, tgt_pose = _sc_split(image2d, pose2d, idxflat)
    return (
        in_img.reshape(B, NIN, 3, 256, 256),
        in_pose.reshape(B, NIN, 4, 4),
        tgt_img.reshape(B, NTGT, 3, 256, 256),
        tgt_pose.reshape(B, NTGT, 4, 4),
    )


# speedup vs baseline: 9.8306x; 9.8089x over previous
import jax
import jax.numpy as jnp
from jax import lax
from jax.experimental import pallas as pl
from jax.experimental.pallas import tpu as pltpu
from jax.experimental.pallas import tpu_sc as plsc

B = 8
V = 16
NIN = 4
NTGT = 8
IMG_W = 3 * 256 * 256
POSE_W = 16
NCH = 4
CH = IMG_W // NCH

_mesh = plsc.VectorSubcoreMesh(core_axis_name="c", subcore_axis_name="s")


def _sc_split(image2d, pose2d, idxflat):
    def body(img_ref, pose_ref, idx_hbm, in_img, tgt_img, in_pose, tgt_pose,
             idx_v, buf0, buf1, sem, isem0, isem1, osem0, osem1):
        cid = lax.axis_index("c")
        sid = lax.axis_index("s")
        w = sid * 2 + cid

        pltpu.sync_copy(idx_hbm, idx_v.at[pl.ds(0, B * NTGT)])

        e0 = 2 * w
        vec = idx_v[pl.ds(e0, 16)]
        v0 = vec[0]
        v1 = vec[1]
        b0 = e0 // NTGT
        b1 = (e0 + 1) // NTGT
        b_in = w // 4
        v_in = w % 4

        pose_handles = [
            pltpu.async_copy(pose_ref.at[b_in * V + v_in], in_pose.at[w], sem),
            pltpu.async_copy(pose_ref.at[b0 * V + v0], tgt_pose.at[e0], sem),
            pltpu.async_copy(pose_ref.at[b1 * V + v1], tgt_pose.at[e0 + 1], sem),
        ]

        tasks = []
        for src_row, dst_ref, dst_row in (
                (b_in * V + v_in, in_img, w),
                (b0 * V + v0, tgt_img, e0),
                (b1 * V + v1, tgt_img, e0 + 1)):
            for c in range(NCH):
                tasks.append((src_row, dst_ref, dst_row, c * CH))

        bufs = (buf0, buf1)
        in_sems = (isem0, isem1)
        out_sems = (osem0, osem1)
        out_handles = [None, None]
        for i, (src_row, dst_ref, dst_row, off) in enumerate(tasks):
            k = i % 2
            if i >= 2:
                out_handles[k].wait()
            pltpu.async_copy(
                img_ref.at[src_row, pl.ds(off, CH)], bufs[k], in_sems[k],
            ).wait()
            out_handles[k] = pltpu.async_copy(
                bufs[k], dst_ref.at[dst_row, pl.ds(off, CH)], out_sems[k])
        out_handles[0].wait()
        out_handles[1].wait()
        for h in pose_handles:
            h.wait()

    f = pl.kernel(
        body,
        out_type=[
            jax.ShapeDtypeStruct((B * NIN, IMG_W), jnp.float32),
            jax.ShapeDtypeStruct((B * NTGT, IMG_W), jnp.float32),
            jax.ShapeDtypeStruct((B * NIN, POSE_W), jnp.float32),
            jax.ShapeDtypeStruct((B * NTGT, POSE_W), jnp.float32),
        ],
        mesh=_mesh,
        scratch_types=[
            pltpu.VMEM((B * NTGT + 16,), jnp.int32),
            pltpu.VMEM((CH,), jnp.float32),
            pltpu.VMEM((CH,), jnp.float32),
            pltpu.SemaphoreType.DMA,
            pltpu.SemaphoreType.DMA,
            pltpu.SemaphoreType.DMA,
            pltpu.SemaphoreType.DMA,
            pltpu.SemaphoreType.DMA,
        ],
    )
    return f(image2d, pose2d, idxflat)


def kernel(image, pose, index):
    image2d = image.reshape(B * V, IMG_W)
    pose2d = pose.reshape(B * V, POSE_W)
    idxflat = index.reshape(B * NTGT).astype(jnp.int32)
    in_img, tgt_img, in_pose, tgt_pose = _sc_split(image2d, pose2d, idxflat)
    return (
        in_img.reshape(B, NIN, 3, 256, 256),
        in_pose.reshape(B, NIN, 4, 4),
        tgt_img.reshape(B, NTGT, 3, 256, 256),
        tgt_pose.reshape(B, NTGT, 4, 4),
    )


# speedup vs baseline: 9.8453x; 1.0015x over previous
import jax
import jax.numpy as jnp
from jax import lax
from jax.experimental import pallas as pl
from jax.experimental.pallas import tpu as pltpu
from jax.experimental.pallas import tpu_sc as plsc

B = 8
V = 16
NIN = 4
NTGT = 8
IMG_W = 3 * 256 * 256
POSE_W = 16
NCH = 8
CH = IMG_W // NCH
NBUF = 4

_mesh = plsc.VectorSubcoreMesh(core_axis_name="c", subcore_axis_name="s")


def _sc_split(image2d, pose2d, idxflat):
    def body(img_ref, pose_ref, idx_hbm, in_img, tgt_img, in_pose, tgt_pose,
             idx_v, *ring):
        bufs = ring[:NBUF]
        sem = ring[NBUF]
        isems = ring[NBUF + 1:2 * NBUF + 1]
        osems = ring[2 * NBUF + 1:]
        cid = lax.axis_index("c")
        sid = lax.axis_index("s")
        w = sid * 2 + cid

        pltpu.sync_copy(idx_hbm, idx_v.at[pl.ds(0, B * NTGT)])

        e0 = 2 * w
        vec = idx_v[pl.ds(e0, 16)]
        v0 = vec[0]
        v1 = vec[1]
        b0 = e0 // NTGT
        b1 = (e0 + 1) // NTGT
        b_in = w // 4
        v_in = w % 4

        pose_handles = [
            pltpu.async_copy(pose_ref.at[b_in * V + v_in], in_pose.at[w], sem),
            pltpu.async_copy(pose_ref.at[b0 * V + v0], tgt_pose.at[e0], sem),
            pltpu.async_copy(pose_ref.at[b1 * V + v1], tgt_pose.at[e0 + 1], sem),
        ]

        tasks = []
        for src_row, dst_ref, dst_row in (
                (b_in * V + v_in, in_img, w),
                (b0 * V + v0, tgt_img, e0),
                (b1 * V + v1, tgt_img, e0 + 1)):
            for c in range(NCH):
                tasks.append((src_row, dst_ref, dst_row, c * CH))
        n = len(tasks)

        in_h = [None] * NBUF
        out_h = [None] * NBUF
        for i in range(-(NBUF - 1), n):
            j = i + NBUF - 1
            if j < n:
                k2 = j % NBUF
                if j >= NBUF:
                    out_h[k2].wait()
                src_row, dst_ref, dst_row, off = tasks[j]
                in_h[k2] = pltpu.async_copy(
                    img_ref.at[src_row, pl.ds(off, CH)], bufs[k2], isems[k2])
            if i >= 0:
                k = i % NBUF
                in_h[k].wait()
                src_row, dst_ref, dst_row, off = tasks[i]
                out_h[k] = pltpu.async_copy(
                    bufs[k], dst_ref.at[dst_row, pl.ds(off, CH)], osems[k])
        for k in range(NBUF):
            out_h[k].wait()
        for h in pose_handles:
            h.wait()

    f = pl.kernel(
        body,
        out_type=[
            jax.ShapeDtypeStruct((B * NIN, IMG_W), jnp.float32),
            jax.ShapeDtypeStruct((B * NTGT, IMG_W), jnp.float32),
            jax.ShapeDtypeStruct((B * NIN, POSE_W), jnp.float32),
            jax.ShapeDtypeStruct((B * NTGT, POSE_W), jnp.float32),
        ],
        mesh=_mesh,
        scratch_types=[
            pltpu.VMEM((B * NTGT + 16,), jnp.int32),
            *[pltpu.VMEM((CH,), jnp.float32) for _ in range(NBUF)],
            *[pltpu.SemaphoreType.DMA for _ in range(2 * NBUF + 1)],
        ],
    )
    return f(image2d, pose2d, idxflat)


def kernel(image, pose, index):
    image2d = image.reshape(B * V, IMG_W)
    pose2d = pose.reshape(B * V, POSE_W)
    idxflat = index.reshape(B * NTGT).astype(jnp.int32)
    in_img, tgt_img, in_pose, tgt_pose = _sc_split(image2d, pose2d, idxflat)
    return (
        in_img.reshape(B, NIN, 3, 256, 256),
        in_pose.reshape(B, NIN, 4, 4),
        tgt_img.reshape(B, NTGT, 3, 256, 256),
        tgt_pose.reshape(B, NTGT, 4, 4),
    )


# speedup vs baseline: 31.7900x; 3.2290x over previous
import jax
import jax.numpy as jnp
from jax import lax
from jax.experimental import pallas as pl
from jax.experimental.pallas import tpu as pltpu
from jax.experimental.pallas import tpu_sc as plsc

B = 8
V = 16
NIN = 4
NTGT = 8
POSE_W = 16
HH = 128
CPC = 256 // HH
NCH = 3 * CPC
NBUF = 3

_mesh = plsc.VectorSubcoreMesh(core_axis_name="c", subcore_axis_name="s")


def _sc_split(image, pose_f, idxflat):
    def body(img_ref, pose_ref, idx_hbm, in_img, tgt_img, in_pose, tgt_pose,
             idx_v, pbuf, *ring):
        bufs = ring[:NBUF]
        sem = ring[NBUF]
        isems = ring[NBUF + 1:2 * NBUF + 1]
        osems = ring[2 * NBUF + 1:]

        cid = lax.axis_index("c")
        sid = lax.axis_index("s")
        w = sid * 2 + cid

        pltpu.sync_copy(idx_hbm, idx_v.at[pl.ds(0, B * NTGT)])

        e0 = 2 * w
        vec = idx_v[pl.ds(e0, 16)]
        v0 = vec[0]
        v1 = vec[1]
        b0 = e0 // NTGT
        b1 = (e0 + 1) // NTGT
        t0 = e0 % NTGT
        t1 = (e0 + 1) % NTGT
        b_in = w // 4
        v_in = w % 4

        _POSE_TASKS = (
            (b_in * V + v_in, in_pose, w),
            (b0 * V + v0, tgt_pose, e0),
            (b1 * V + v1, tgt_pose, e0 + 1),
        )
        for r, (src_row, _, _) in enumerate(_POSE_TASKS):
            pltpu.async_copy(
                pose_ref.at[pl.ds(src_row * POSE_W, POSE_W)],
                pbuf.at[pl.ds(r * POSE_W, POSE_W)], sem).wait()
        pose_handles = [
            pltpu.async_copy(
                pbuf.at[pl.ds(r * POSE_W, POSE_W)],
                dst_ref.at[pl.ds(dst_row * POSE_W, POSE_W)], sem)
            for r, (_, dst_ref, dst_row) in enumerate(_POSE_TASKS)
        ]

        tasks = []
        for src_b, src_v, dst_ref, dst_b, dst_v in (
                (b_in, v_in, in_img, b_in, v_in),
                (b0, v0, tgt_img, b0, t0),
                (b1, v1, tgt_img, b1, t1)):
            for c in range(NCH):
                ci, h0 = c // CPC, (c % CPC) * HH
                tasks.append((src_b, src_v, dst_ref, dst_b, dst_v, ci, h0))
        n = len(tasks)

        in_h = [None] * NBUF
        out_h = [None] * NBUF
        for i in range(-(NBUF - 1), n):
            j = i + NBUF - 1
            if j < n:
                k2 = j % NBUF
                if j >= NBUF:
                    out_h[k2].wait()
                sb, sv, _, _, _, ci, h0 = tasks[j]
                in_h[k2] = pltpu.async_copy(
                    img_ref.at[sb, sv, ci, pl.ds(h0, HH)], bufs[k2], isems[k2])
            if i >= 0:
                k = i % NBUF
                in_h[k].wait()
                _, _, dst_ref, db, dv, ci, h0 = tasks[i]
                out_h[k] = pltpu.async_copy(
                    bufs[k], dst_ref.at[db, dv, ci, pl.ds(h0, HH)], osems[k])
        for k in range(NBUF):
            out_h[k].wait()
        for h in pose_handles:
            h.wait()

    f = pl.kernel(
        body,
        out_type=[
            jax.ShapeDtypeStruct((B, NIN, 3, 256, 256), jnp.float32),
            jax.ShapeDtypeStruct((B, NTGT, 3, 256, 256), jnp.float32),
            jax.ShapeDtypeStruct((B * NIN * POSE_W,), jnp.float32),
            jax.ShapeDtypeStruct((B * NTGT * POSE_W,), jnp.float32),
        ],
        mesh=_mesh,
        compiler_params=pltpu.CompilerParams(use_tc_tiling_on_sc=True),
        scratch_types=[
            pltpu.VMEM((B * NTGT + 16,), jnp.int32),
            pltpu.VMEM((3 * POSE_W,), jnp.float32),
            *[pltpu.VMEM((HH, 256), jnp.float32) for _ in range(NBUF)],
            *[pltpu.SemaphoreType.DMA for _ in range(2 * NBUF + 1)],
        ],
    )
    return f(image, pose_f, idxflat)


def kernel(image, pose, index):
    pose_f = pose.reshape(B * V * POSE_W)
    idxflat = index.reshape(B * NTGT).astype(jnp.int32)
    in_img, tgt_img, in_pose, tgt_pose = _sc_split(image, pose_f, idxflat)
    return (
        in_img,
        in_pose.reshape(B, NIN, 4, 4),
        tgt_img,
        tgt_pose.reshape(B, NTGT, 4, 4),
    )


# speedup vs baseline: 33.0558x; 1.0398x over previous
import jax
import jax.numpy as jnp
from jax import lax
from jax.experimental import pallas as pl
from jax.experimental.pallas import tpu as pltpu
from jax.experimental.pallas import tpu_sc as plsc

B = 8
V = 16
NIN = 4
NTGT = 8
POSE_W = 16
HH = 128
CPC = 256 // HH
NCH = 3 * CPC
NBUF = 3

_mesh = plsc.VectorSubcoreMesh(core_axis_name="c", subcore_axis_name="s")


def _sc_split(image, pose_f, idxflat):
    def body(img_ref, pose_ref, idx_hbm, tgt_img, in_pose, tgt_pose,
             idx_v, pbuf, *ring):
        bufs = ring[:NBUF]
        sem = ring[NBUF]
        isems = ring[NBUF + 1:2 * NBUF + 1]
        osems = ring[2 * NBUF + 1:]

        cid = lax.axis_index("c")
        sid = lax.axis_index("s")
        w = sid * 2 + cid

        pltpu.sync_copy(idx_hbm, idx_v.at[pl.ds(0, B * NTGT)])

        e0 = 2 * w
        vec = idx_v[pl.ds(e0, 16)]
        v0 = vec[0]
        v1 = vec[1]
        b0 = e0 // NTGT
        b1 = (e0 + 1) // NTGT
        t0 = e0 % NTGT
        t1 = (e0 + 1) % NTGT
        b_in = w // 4
        v_in = w % 4

        _POSE_TASKS = (
            (b_in * V + v_in, in_pose, w),
            (b0 * V + v0, tgt_pose, e0),
            (b1 * V + v1, tgt_pose, e0 + 1),
        )
        for r, (src_row, _, _) in enumerate(_POSE_TASKS):
            pltpu.async_copy(
                pose_ref.at[pl.ds(src_row * POSE_W, POSE_W)],
                pbuf.at[pl.ds(r * POSE_W, POSE_W)], sem).wait()
        pose_handles = [
            pltpu.async_copy(
                pbuf.at[pl.ds(r * POSE_W, POSE_W)],
                dst_ref.at[pl.ds(dst_row * POSE_W, POSE_W)], sem)
            for r, (_, dst_ref, dst_row) in enumerate(_POSE_TASKS)
        ]

        tasks = []
        for src_b, src_v, dst_ref, dst_b, dst_v in (
                (b0, v0, tgt_img, b0, t0),
                (b1, v1, tgt_img, b1, t1)):
            for c in range(NCH):
                ci, h0 = c // CPC, (c % CPC) * HH
                tasks.append((src_b, src_v, dst_ref, dst_b, dst_v, ci, h0))
        n = len(tasks)

        in_h = [None] * NBUF
        out_h = [None] * NBUF
        for i in range(-(NBUF - 1), n):
            j = i + NBUF - 1
            if j < n:
                k2 = j % NBUF
                if j >= NBUF:
                    out_h[k2].wait()
                sb, sv, _, _, _, ci, h0 = tasks[j]
                in_h[k2] = pltpu.async_copy(
                    img_ref.at[sb, sv, ci, pl.ds(h0, HH)], bufs[k2], isems[k2])
            if i >= 0:
                k = i % NBUF
                in_h[k].wait()
                _, _, dst_ref, db, dv, ci, h0 = tasks[i]
                out_h[k] = pltpu.async_copy(
                    bufs[k], dst_ref.at[db, dv, ci, pl.ds(h0, HH)], osems[k])
        for k in range(NBUF):
            out_h[k].wait()
        for h in pose_handles:
            h.wait()

    f = pl.kernel(
        body,
        out_type=[
            jax.ShapeDtypeStruct((B, NTGT, 3, 256, 256), jnp.float32),
            jax.ShapeDtypeStruct((B * NIN * POSE_W,), jnp.float32),
            jax.ShapeDtypeStruct((B * NTGT * POSE_W,), jnp.float32),
        ],
        mesh=_mesh,
        compiler_params=pltpu.CompilerParams(use_tc_tiling_on_sc=True),
        scratch_types=[
            pltpu.VMEM((B * NTGT + 16,), jnp.int32),
            pltpu.VMEM((3 * POSE_W,), jnp.float32),
            *[pltpu.VMEM((HH, 256), jnp.float32) for _ in range(NBUF)],
            *[pltpu.SemaphoreType.DMA for _ in range(2 * NBUF + 1)],
        ],
    )
    return f(image, pose_f, idxflat)


def _tc_slice(image):
    def body(img_blk, out_blk):
        out_blk[...] = img_blk[...]

    return pl.pallas_call(
        body,
        grid=(B * NIN,),
        in_specs=[pl.BlockSpec(
            (1, 1, 3, 256, 256), lambda i: (i // NIN, i % NIN, 0, 0, 0))],
        out_specs=pl.BlockSpec(
            (1, 1, 3, 256, 256), lambda i: (i // NIN, i % NIN, 0, 0, 0)),
        out_shape=jax.ShapeDtypeStruct((B, NIN, 3, 256, 256), jnp.float32),
    )(image)


def kernel(image, pose, index):
    pose_f = pose.reshape(B * V * POSE_W)
    idxflat = index.reshape(B * NTGT).astype(jnp.int32)
    tgt_img, in_pose, tgt_pose = _sc_split(image, pose_f, idxflat)
    in_img = _tc_slice(image)
    return (
        in_img,
        in_pose.reshape(B, NIN, 4, 4),
        tgt_img,
        tgt_pose.reshape(B, NTGT, 4, 4),
    )


# speedup vs baseline: 33.4511x; 1.0120x over previous
import jax
import jax.numpy as jnp
from jax import lax
from jax.experimental import pallas as pl
from jax.experimental.pallas import tpu as pltpu
from jax.experimental.pallas import tpu_sc as plsc

B = 8
V = 16
NIN = 4
NTGT = 8
POSE_W = 16
HH = 128
CPC = 256 // HH
NCH = 3 * CPC
NBUF = 3

_mesh = plsc.VectorSubcoreMesh(core_axis_name="c", subcore_axis_name="s")


def _sc_split(image, pose_f, idxflat):
    def body(img_ref, pose_ref, idx_hbm, tgt_img, in_pose, tgt_pose,
             idx_v, pbuf, *ring):
        bufs = ring[:NBUF]
        sem = ring[NBUF]
        isems = ring[NBUF + 1:2 * NBUF + 1]
        osems = ring[2 * NBUF + 1:]

        cid = lax.axis_index("c")
        sid = lax.axis_index("s")
        w = sid * 2 + cid

        pltpu.sync_copy(idx_hbm, idx_v.at[pl.ds(0, B * NTGT)])

        e0 = 2 * w
        vec = idx_v[pl.ds(e0, 16)]
        v0 = vec[0]
        v1 = vec[1]
        b0 = e0 // NTGT
        b1 = (e0 + 1) // NTGT
        t0 = e0 % NTGT
        t1 = (e0 + 1) % NTGT
        b_in = w // 4
        v_in = w % 4

        tasks = []
        for src_b, src_v, dst_ref, dst_b, dst_v in (
                (b0, v0, tgt_img, b0, t0),
                (b1, v1, tgt_img, b1, t1)):
            for c in range(NCH):
                ci, h0 = c // CPC, (c % CPC) * HH
                tasks.append((src_b, src_v, dst_ref, dst_b, dst_v, ci, h0))
        n = len(tasks)

        in_h = [None] * NBUF
        out_h = [None] * NBUF
        for i in range(-(NBUF - 1), n):
            j = i + NBUF - 1
            if j < n:
                k2 = j % NBUF
                if j >= NBUF:
                    out_h[k2].wait()
                sb, sv, _, _, _, ci, h0 = tasks[j]
                in_h[k2] = pltpu.async_copy(
                    img_ref.at[sb, sv, ci, pl.ds(h0, HH)], bufs[k2], isems[k2])
            if i >= 0:
                k = i % NBUF
                in_h[k].wait()
                _, _, dst_ref, db, dv, ci, h0 = tasks[i]
                out_h[k] = pltpu.async_copy(
                    bufs[k], dst_ref.at[db, dv, ci, pl.ds(h0, HH)], osems[k])

        pose_tasks = (
            (b_in * V + v_in, in_pose, w),
            (b0 * V + v0, tgt_pose, e0),
            (b1 * V + v1, tgt_pose, e0 + 1),
        )
        for r, (src_row, _, _) in enumerate(pose_tasks):
            pltpu.async_copy(
                pose_ref.at[pl.ds(src_row * POSE_W, POSE_W)],
                pbuf.at[pl.ds(r * POSE_W, POSE_W)], sem).wait()
        pose_handles = [
            pltpu.async_copy(
                pbuf.at[pl.ds(r * POSE_W, POSE_W)],
                dst_ref.at[pl.ds(dst_row * POSE_W, POSE_W)], sem)
            for r, (_, dst_ref, dst_row) in enumerate(pose_tasks)
        ]

        for k in range(NBUF):
            out_h[k].wait()
        for h in pose_handles:
            h.wait()

    f = pl.kernel(
        body,
        out_type=[
            jax.ShapeDtypeStruct((B, NTGT, 3, 256, 256), jnp.float32),
            jax.ShapeDtypeStruct((B * NIN * POSE_W,), jnp.float32),
            jax.ShapeDtypeStruct((B * NTGT * POSE_W,), jnp.float32),
        ],
        mesh=_mesh,
        compiler_params=pltpu.CompilerParams(use_tc_tiling_on_sc=True),
        scratch_types=[
            pltpu.VMEM((B * NTGT + 16,), jnp.int32),
            pltpu.VMEM((3 * POSE_W,), jnp.float32),
            *[pltpu.VMEM((HH, 256), jnp.float32) for _ in range(NBUF)],
            *[pltpu.SemaphoreType.DMA for _ in range(2 * NBUF + 1)],
        ],
    )
    return f(image, pose_f, idxflat)


def _tc_slice(image):
    def body(img_blk, out_blk):
        out_blk[...] = img_blk[...]

    return pl.pallas_call(
        body,
        grid=(B * NIN,),
        in_specs=[pl.BlockSpec(
            (1, 1, 3, 256, 256), lambda i: (i // NIN, i % NIN, 0, 0, 0))],
        out_specs=pl.BlockSpec(
            (1, 1, 3, 256, 256), lambda i: (i // NIN, i % NIN, 0, 0, 0)),
        out_shape=jax.ShapeDtypeStruct((B, NIN, 3, 256, 256), jnp.float32),
    )(image)


def kernel(image, pose, index):
    pose_f = pose.reshape(B * V * POSE_W)
    idxflat = index.reshape(B * NTGT).astype(jnp.int32)
    in_img = _tc_slice(image)
    tgt_img, in_pose, tgt_pose = _sc_split(image, pose_f, idxflat)
    return (
        in_img,
        in_pose.reshape(B, NIN, 4, 4),
        tgt_img,
        tgt_pose.reshape(B, NTGT, 4, 4),
    )


# speedup vs baseline: 34.0681x; 1.0184x over previous
import jax
import jax.numpy as jnp
from jax import lax
from jax.experimental import pallas as pl
from jax.experimental.pallas import tpu as pltpu
from jax.experimental.pallas import tpu_sc as plsc

B = 8
V = 16
NIN = 4
NTGT = 8
POSE_W = 16
HH = 128
CPC = 256 // HH
NCH = 3 * CPC
NBUF = 3

_mesh = plsc.VectorSubcoreMesh(core_axis_name="c", subcore_axis_name="s")


def _sc_split(image, idxflat):
    def body(img_ref, idx_hbm, tgt_img, idx_v, *ring):
        bufs = ring[:NBUF]
        sem = ring[NBUF]
        isems = ring[NBUF + 1:2 * NBUF + 1]
        osems = ring[2 * NBUF + 1:]

        cid = lax.axis_index("c")
        sid = lax.axis_index("s")
        w = sid * 2 + cid

        pltpu.sync_copy(idx_hbm, idx_v.at[pl.ds(0, B * NTGT)])

        e0 = 2 * w
        vec = idx_v[pl.ds(e0, 16)]
        v0 = vec[0]
        v1 = vec[1]
        b0 = e0 // NTGT
        b1 = (e0 + 1) // NTGT
        t0 = e0 % NTGT
        t1 = (e0 + 1) % NTGT

        tasks = []
        for src_b, src_v, dst_ref, dst_b, dst_v in (
                (b0, v0, tgt_img, b0, t0),
                (b1, v1, tgt_img, b1, t1)):
            for c in range(NCH):
                ci, h0 = c // CPC, (c % CPC) * HH
                tasks.append((src_b, src_v, dst_ref, dst_b, dst_v, ci, h0))
        n = len(tasks)

        in_h = [None] * NBUF
        out_h = [None] * NBUF
        for i in range(-(NBUF - 1), n):
            j = i + NBUF - 1
            if j < n:
                k2 = j % NBUF
                if j >= NBUF:
                    out_h[k2].wait()
                sb, sv, _, _, _, ci, h0 = tasks[j]
                in_h[k2] = pltpu.async_copy(
                    img_ref.at[sb, sv, ci, pl.ds(h0, HH)], bufs[k2], isems[k2])
            if i >= 0:
                k = i % NBUF
                in_h[k].wait()
                _, _, dst_ref, db, dv, ci, h0 = tasks[i]
                out_h[k] = pltpu.async_copy(
                    bufs[k], dst_ref.at[db, dv, ci, pl.ds(h0, HH)], osems[k])
        for k in range(NBUF):
            out_h[k].wait()

    f = pl.kernel(
        body,
        out_type=[
            jax.ShapeDtypeStruct((B, NTGT, 3, 256, 256), jnp.float32),
        ],
        mesh=_mesh,
        compiler_params=pltpu.CompilerParams(use_tc_tiling_on_sc=True),
        scratch_types=[
            pltpu.VMEM((B * NTGT + 16,), jnp.int32),
            *[pltpu.VMEM((HH, 256), jnp.float32) for _ in range(NBUF)],
            *[pltpu.SemaphoreType.DMA for _ in range(2 * NBUF + 1)],
        ],
    )
    return f(image, idxflat)


def _tc_slice(image, pose, idxflat):
    def body(idx_s, img_blk, pose_blk, out_blk, in_pose_blk, tgt_pose_blk):
        out_blk[...] = img_blk[...]

        @pl.when(pl.program_id(0) == 0)
        def _():
            in_pose_blk[...] = pose_blk[:, :NIN]
            for b in range(B):
                for t in range(NTGT):
                    v = idx_s[b * NTGT + t]
                    tgt_pose_blk[b, t] = pose_blk[b, v]

    return pl.pallas_call(
        body,
        grid=(B * NIN,),
        in_specs=[
            pl.BlockSpec(memory_space=pltpu.SMEM),
            pl.BlockSpec(
                (1, 1, 3, 256, 256), lambda i: (i // NIN, i % NIN, 0, 0, 0)),
            pl.BlockSpec((B, V, 4, 4), lambda i: (0, 0, 0, 0)),
        ],
        out_specs=[
            pl.BlockSpec(
                (1, 1, 3, 256, 256), lambda i: (i // NIN, i % NIN, 0, 0, 0)),
            pl.BlockSpec((B, NIN, 4, 4), lambda i: (0, 0, 0, 0)),
            pl.BlockSpec((B, NTGT, 4, 4), lambda i: (0, 0, 0, 0)),
        ],
        out_shape=[
            jax.ShapeDtypeStruct((B, NIN, 3, 256, 256), jnp.float32),
            jax.ShapeDtypeStruct((B, NIN, 4, 4), jnp.float32),
            jax.ShapeDtypeStruct((B, NTGT, 4, 4), jnp.float32),
        ],
    )(idxflat, image, pose)


def kernel(image, pose, index):
    idxflat = index.reshape(B * NTGT).astype(jnp.int32)
    in_img, in_pose, tgt_pose = _tc_slice(image, pose, idxflat)
    (tgt_img,) = _sc_split(image, idxflat)
    return (in_img, in_pose, tgt_img, tgt_pose)
